# baseline (speedup 1.0000x reference)
import functools

import jax
import jax.numpy as jnp
from jax import lax
from jax.experimental import pallas as pl
from jax.experimental.pallas import tpu as pltpu
from jax.experimental.pallas import tpu_sc as plsc

VOCAB = 100000
TOK_DIM = 32
MAX_LEN = 200
POS_DIM = 32
BATCH = 4096

OUT_DIM = TOK_DIM + POS_DIM
NC, NS = 2, 16
NW = NC * NS
BW = BATCH // NW
CL = 4
NCH = MAX_LEN // CL
NBUF = 2
RT = TOK_DIM // 8
RP = POS_DIM // 8


@functools.partial(
    pl.kernel,
    mesh=plsc.VectorSubcoreMesh(core_axis_name="c", subcore_axis_name="s"),
    out_type=jax.ShapeDtypeStruct((MAX_LEN, OUT_DIM // 8, NW, 8, BW),
                                  jnp.float32),
    compiler_params=pltpu.CompilerParams(use_tc_tiling_on_sc=False,
                                         needs_layout_passes=False),
    scratch_types=[
        pltpu.VMEM((NBUF, CL, BW), jnp.int32),
        pltpu.VMEM((NBUF * CL * BW, TOK_DIM), jnp.float32),
        pltpu.VMEM((NBUF, CL, RT, 8, BW), jnp.float32),
        pltpu.SemaphoreType.DMA,
        pltpu.SemaphoreType.DMA,
        pltpu.SemaphoreType.DMA,
        pltpu.SemaphoreType.DMA,
    ],
)
def _emb(tok_hbm, idxt_hbm, posp_hbm, out_hbm, idx_v, rows_v, s_v,
         isem, gsem, ssem, psem):
    w = lax.axis_index("s") * NC + lax.axis_index("c")
    b0 = w * BW

    def idx_load(g, fire):
        buf = lax.rem(g, NBUF)
        d = pltpu.make_async_copy(
            idxt_hbm.at[pl.ds(g * CL, CL), pl.ds(b0, BW)],
            idx_v.at[buf], isem)
        d.start() if fire else d.wait()

    def gather(g, fire):
        buf = lax.rem(g, NBUF)
        for j in range(CL):
            d = pltpu.make_async_copy(
                tok_hbm.at[idx_v.at[buf, j]],
                rows_v.at[pl.ds((buf * CL + j) * BW, BW)], gsem)
            d.start() if fire else d.wait()

    def store(g, fire):
        buf = lax.rem(g, NBUF)
        for r in range(RT):
            d = pltpu.make_async_copy(
                s_v.at[buf, slice(None), r],
                out_hbm.at[pl.ds(g * CL, CL), r, w, slice(None), slice(None)],
                ssem)
            d.start() if fire else d.wait()

    def pos_store(fire):
        for r in range(RP):
            d = pltpu.make_async_copy(
                posp_hbm.at[slice(None), r],
                out_hbm.at[slice(None), RT + r, w, slice(None), slice(None)],
                psem)
            d.start() if fire else d.wait()

    iota16 = lax.iota(jnp.int32, 16)

    dsplats = [jnp.full((16,), d, jnp.int32) for d in range(TOK_DIM)]

    def transpose_chunk(buf):
        for j in range(CL):
            base = (buf * CL + j) * BW

            def tbody(tb, c):
                tidx = iota16 + (base + tb * 16)
                t0 = tb * 16
                vs = [plsc.load_gather(rows_v, [tidx, dsplats[d]])
                      for d in range(TOK_DIM)]
                for d in range(TOK_DIM):
                    s_v[buf, j, d // 8, d % 8, pl.ds(t0, 16)] = vs[d]
                return c

            lax.fori_loop(0, BW // 16, tbody, 0)

    pos_store(True)
    idx_load(0, True)
    idx_load(0, False)
    gather(0, True)
    idx_load(1, True)

    def main(g, carry):
        gather(g, False)
        pl.when(g + 1 < NCH)(lambda: idx_load(g + 1, False))
        pl.when(g + 1 < NCH)(lambda: gather(g + 1, True))
        pl.when(g + 2 < NCH)(lambda: idx_load(g + 2, True))
        pl.when(g >= 2)(lambda: store(g - 2, False))
        transpose_chunk(lax.rem(g, NBUF))
        store(g, True)
        return carry

    lax.fori_loop(0, NCH, main, 0)

    store(NCH - 2, False)
    store(NCH - 1, False)
    pos_store(False)


def kernel(indices, tok_table, pos_table):
    idxt = indices.T.astype(jnp.int32)
    posp = jnp.broadcast_to(
        pos_table.reshape(MAX_LEN, RP, 8, 1),
        (MAX_LEN, RP, 8, BW))
    out5 = _emb(tok_table, idxt, posp)
    return out5.transpose(2, 4, 0, 1, 3).reshape(BATCH, MAX_LEN, OUT_DIM)


# speedup vs baseline: 6.6189x; 6.6189x over previous
import functools

import jax
import jax.numpy as jnp
from jax import lax
from jax.experimental import pallas as pl
from jax.experimental.pallas import tpu as pltpu
from jax.experimental.pallas import tpu_sc as plsc

VOCAB = 100000
TOK_DIM = 32
MAX_LEN = 200
POS_DIM = 32
BATCH = 4096

OUT_DIM = TOK_DIM + POS_DIM
NC, NS = 2, 16
NW = NC * NS
BW = BATCH // NW
CL = 4
NCH = MAX_LEN // CL
NBUF = 2
RT = TOK_DIM // 8
RP = POS_DIM // 8


@functools.partial(
    pl.kernel,
    mesh=plsc.VectorSubcoreMesh(core_axis_name="c", subcore_axis_name="s"),
    out_type=jax.ShapeDtypeStruct((MAX_LEN, OUT_DIM // 8, NW, 8, BW),
                                  jnp.float32),
    compiler_params=pltpu.CompilerParams(use_tc_tiling_on_sc=False,
                                         needs_layout_passes=False),
    scratch_types=[
        pltpu.VMEM((NBUF, CL, BW), jnp.int32),
        pltpu.VMEM((NBUF * CL * BW, TOK_DIM), jnp.float32),
        pltpu.VMEM((NBUF, CL, RT, 8, BW), jnp.float32),
        pltpu.VMEM((NBUF, CL, POS_DIM, BW), jnp.float32),
        pltpu.SemaphoreType.DMA,
        pltpu.SemaphoreType.DMA,
        pltpu.SemaphoreType.DMA,
        pltpu.SemaphoreType.DMA,
        pltpu.SemaphoreType.DMA,
    ],
)
def _emb(tok_hbm, idxt_hbm, posp_hbm, out_hbm, idx_v, rows_v, s_v, pos_v,
         isem, gsem, ssem, psem, plsem):
    w = lax.axis_index("s") * NC + lax.axis_index("c")
    b0 = w * BW

    def idx_load(g, fire):
        buf = lax.rem(g, NBUF)
        d = pltpu.make_async_copy(
            idxt_hbm.at[pl.ds(g * CL, CL), pl.ds(b0, BW)],
            idx_v.at[buf], isem)
        d.start() if fire else d.wait()

    def gather(g, fire):
        buf = lax.rem(g, NBUF)
        for j in range(CL):
            d = pltpu.make_async_copy(
                tok_hbm.at[idx_v.at[buf, j]],
                rows_v.at[pl.ds((buf * CL + j) * BW, BW)], gsem)
            d.start() if fire else d.wait()

    def store(g, fire):
        buf = lax.rem(g, NBUF)
        for r in range(RT):
            d = pltpu.make_async_copy(
                s_v.at[buf, slice(None), r],
                out_hbm.at[pl.ds(g * CL, CL), r, w, slice(None), slice(None)],
                ssem)
            d.start() if fire else d.wait()

    def pos_load(g, fire):
        buf = lax.rem(g, NBUF)
        d = pltpu.make_async_copy(
            posp_hbm.at[pl.ds(g * CL, CL)], pos_v.at[buf], plsem)
        d.start() if fire else d.wait()

    def pos_out(g, fire):
        buf = lax.rem(g, NBUF)
        for r in range(RP):
            d = pltpu.make_async_copy(
                pos_v.at[buf, slice(None), pl.ds(r * 8, 8)],
                out_hbm.at[pl.ds(g * CL, CL), RT + r, w,
                           slice(None), slice(None)],
                psem)
            d.start() if fire else d.wait()

    iota16 = lax.iota(jnp.int32, 16)

    dsplats = [jnp.full((16,), d, jnp.int32) for d in range(TOK_DIM)]

    def transpose_chunk(buf):
        for j in range(CL):
            base = (buf * CL + j) * BW

            def tbody(tb, c):
                tidx = iota16 + (base + tb * 16)
                t0 = tb * 16
                vs = [plsc.load_gather(rows_v, [tidx, dsplats[d]])
                      for d in range(TOK_DIM)]
                for d in range(TOK_DIM):
                    s_v[buf, j, d // 8, d % 8, pl.ds(t0, 16)] = vs[d]
                return c

            lax.fori_loop(0, BW // 16, tbody, 0)

    idx_load(0, True)
    pos_load(0, True)
    idx_load(0, False)
    gather(0, True)
    idx_load(1, True)

    def main(g, carry):
        gather(g, False)
        pl.when(g + 1 < NCH)(lambda: idx_load(g + 1, False))
        pl.when(g + 1 < NCH)(lambda: gather(g + 1, True))
        pl.when(g + 2 < NCH)(lambda: idx_load(g + 2, True))
        pl.when(g >= 1)(lambda: pos_out(g - 1, False))
        pl.when(g + 1 < NCH)(lambda: pos_load(g + 1, True))
        pl.when(g >= 2)(lambda: store(g - 2, False))
        transpose_chunk(lax.rem(g, NBUF))
        store(g, True)
        pos_load(g, False)
        pos_out(g, True)
        return carry

    lax.fori_loop(0, NCH, main, 0)

    store(NCH - 2, False)
    store(NCH - 1, False)
    pos_out(NCH - 1, False)


def kernel(indices, tok_table, pos_table):
    idxt = indices.T.astype(jnp.int32)
    posp = jnp.broadcast_to(
        pos_table.reshape(MAX_LEN, POS_DIM, 1),
        (MAX_LEN, POS_DIM, BW))
    out5 = _emb(tok_table, idxt, posp)
    return out5.transpose(2, 4, 0, 1, 3).reshape(BATCH, MAX_LEN, OUT_DIM)


# speedup vs baseline: 6.9167x; 1.0450x over previous
import functools

import jax
import jax.numpy as jnp
from jax import lax
from jax.experimental import pallas as pl
from jax.experimental.pallas import tpu as pltpu
from jax.experimental.pallas import tpu_sc as plsc

VOCAB = 100000
TOK_DIM = 32
MAX_LEN = 200
POS_DIM = 32
BATCH = 4096

OUT_DIM = TOK_DIM + POS_DIM
NC, NS = 2, 16
NW = NC * NS
BW = BATCH // NW
CL = 4
NCH = MAX_LEN // CL
NBUF = 2
RT = TOK_DIM // 8
RP = POS_DIM // 8


@functools.partial(
    pl.kernel,
    mesh=plsc.VectorSubcoreMesh(core_axis_name="c", subcore_axis_name="s"),
    out_type=jax.ShapeDtypeStruct((MAX_LEN, OUT_DIM // 8, NW, 8, BW),
                                  jnp.float32),
    compiler_params=pltpu.CompilerParams(use_tc_tiling_on_sc=False,
                                         needs_layout_passes=False),
    scratch_types=[
        pltpu.VMEM((NBUF, CL, BW), jnp.int32),
        pltpu.VMEM((NBUF * CL * BW, TOK_DIM), jnp.float32),
        pltpu.VMEM((NBUF, CL, RT, 8, BW), jnp.float32),
        pltpu.VMEM_SHARED((MAX_LEN, POS_DIM, BW), jnp.float32),
        pltpu.SemaphoreType.DMA,
        pltpu.SemaphoreType.DMA,
        pltpu.SemaphoreType.DMA,
        pltpu.SemaphoreType.DMA,
    ],
)
def _emb(tok_hbm, idxt_hbm, posp_hbm, out_hbm, idx_v, rows_v, s_v, shpos,
         isem, gsem, ssem, psem):
    w = lax.axis_index("s") * NC + lax.axis_index("c")
    b0 = w * BW

    def idx_load(g, fire):
        buf = lax.rem(g, NBUF)
        d = pltpu.make_async_copy(
            idxt_hbm.at[pl.ds(g * CL, CL), pl.ds(b0, BW)],
            idx_v.at[buf], isem)
        d.start() if fire else d.wait()

    def gather(g, fire):
        buf = lax.rem(g, NBUF)
        for j in range(CL):
            d = pltpu.make_async_copy(
                tok_hbm.at[idx_v.at[buf, j]],
                rows_v.at[pl.ds((buf * CL + j) * BW, BW)], gsem)
            d.start() if fire else d.wait()

    def store(g, fire):
        buf = lax.rem(g, NBUF)
        for r in range(RT):
            d = pltpu.make_async_copy(
                s_v.at[buf, slice(None), r],
                out_hbm.at[pl.ds(g * CL, CL), r, w, slice(None), slice(None)],
                ssem)
            d.start() if fire else d.wait()

    def pos_out(g, fire):
        for r in range(RP):
            d = pltpu.make_async_copy(
                shpos.at[pl.ds(g * CL, CL), pl.ds(r * 8, 8)],
                out_hbm.at[pl.ds(g * CL, CL), RT + r, w,
                           slice(None), slice(None)],
                psem)
            d.start() if fire else d.wait()

    iota16 = lax.iota(jnp.int32, 16)

    dsplats = [jnp.full((16,), d, jnp.int32) for d in range(TOK_DIM)]

    def transpose_chunk(buf):
        for j in range(CL):
            base = (buf * CL + j) * BW

            def tbody(tb, c):
                tidx = iota16 + (base + tb * 16)
                t0 = tb * 16
                vs = [plsc.load_gather(rows_v, [tidx, dsplats[d]])
                      for d in range(TOK_DIM)]
                for d in range(TOK_DIM):
                    s_v[buf, j, d // 8, d % 8, pl.ds(t0, 16)] = vs[d]
                return c

            lax.fori_loop(0, BW // 16, tbody, 0)

    idx_load(0, True)
    pl.when(lax.axis_index("s") == 0)(
        lambda: pltpu.sync_copy(posp_hbm, shpos))
    idx_load(0, False)
    gather(0, True)
    idx_load(1, True)
    plsc.subcore_barrier()

    def main(g, carry):
        gather(g, False)
        pl.when(g + 1 < NCH)(lambda: idx_load(g + 1, False))
        pl.when(g + 1 < NCH)(lambda: gather(g + 1, True))
        pl.when(g + 2 < NCH)(lambda: idx_load(g + 2, True))
        pl.when(g >= 1)(lambda: pos_out(g - 1, False))
        pl.when(g >= 2)(lambda: store(g - 2, False))
        transpose_chunk(lax.rem(g, NBUF))
        store(g, True)
        pos_out(g, True)
        return carry

    lax.fori_loop(0, NCH, main, 0)

    store(NCH - 2, False)
    store(NCH - 1, False)
    pos_out(NCH - 1, False)


def kernel(indices, tok_table, pos_table):
    idxt = indices.T.astype(jnp.int32)
    posp = jnp.broadcast_to(
        pos_table.reshape(MAX_LEN, POS_DIM, 1),
        (MAX_LEN, POS_DIM, BW))
    out5 = _emb(tok_table, idxt, posp)
    return out5.transpose(2, 4, 0, 1, 3).reshape(BATCH, MAX_LEN, OUT_DIM)


# speedup vs baseline: 13.2583x; 1.9168x over previous
import functools

import jax
import jax.numpy as jnp
from jax import lax
from jax.experimental import pallas as pl
from jax.experimental.pallas import tpu as pltpu
from jax.experimental.pallas import tpu_sc as plsc

VOCAB = 100000
TOK_DIM = 32
MAX_LEN = 200
POS_DIM = 32
BATCH = 4096

OUT_DIM = TOK_DIM + POS_DIM
NC, NS = 2, 16
NW = NC * NS
BW = BATCH // NW
CL = 4
NCH = MAX_LEN // CL
NBUF = 2
RT = TOK_DIM // 8
RP = POS_DIM // 8


@functools.partial(
    pl.kernel,
    mesh=plsc.VectorSubcoreMesh(core_axis_name="c", subcore_axis_name="s"),
    out_type=jax.ShapeDtypeStruct((MAX_LEN, OUT_DIM // 8, NW, 8, BW),
                                  jnp.float32),
    compiler_params=pltpu.CompilerParams(use_tc_tiling_on_sc=False,
                                         needs_layout_passes=False),
    scratch_types=[
        pltpu.VMEM((NBUF, CL, BW), jnp.int32),
        pltpu.VMEM((NBUF * CL * BW, TOK_DIM + 8), jnp.float32),
        pltpu.VMEM((NBUF, CL, RT, 8, BW), jnp.float32),
        pltpu.VMEM_SHARED((MAX_LEN, POS_DIM, BW), jnp.float32),
        pltpu.SemaphoreType.DMA,
        pltpu.SemaphoreType.DMA,
        pltpu.SemaphoreType.DMA,
        pltpu.SemaphoreType.DMA,
    ],
)
def _emb(tok_hbm, idxt_hbm, posp_hbm, out_hbm, idx_v, rows_v, s_v, shpos,
         isem, gsem, ssem, psem):
    w = lax.axis_index("s") * NC + lax.axis_index("c")
    b0 = w * BW

    def idx_load(g, fire):
        buf = lax.rem(g, NBUF)
        d = pltpu.make_async_copy(
            idxt_hbm.at[pl.ds(g * CL, CL), pl.ds(b0, BW)],
            idx_v.at[buf], isem)
        d.start() if fire else d.wait()

    def gather(g, fire):
        buf = lax.rem(g, NBUF)
        for j in range(CL):
            d = pltpu.make_async_copy(
                tok_hbm.at[idx_v.at[buf, j]],
                rows_v.at[pl.ds((buf * CL + j) * BW, BW)], gsem)
            d.start() if fire else d.wait()

    def store(g, fire):
        buf = lax.rem(g, NBUF)
        for r in range(RT):
            d = pltpu.make_async_copy(
                s_v.at[buf, slice(None), r],
                out_hbm.at[pl.ds(g * CL, CL), r, w, slice(None), slice(None)],
                ssem)
            d.start() if fire else d.wait()

    def pos_out(g, fire):
        for r in range(RP):
            d = pltpu.make_async_copy(
                shpos.at[pl.ds(g * CL, CL), pl.ds(r * 8, 8)],
                out_hbm.at[pl.ds(g * CL, CL), RT + r, w,
                           slice(None), slice(None)],
                psem)
            d.start() if fire else d.wait()

    iota16 = lax.iota(jnp.int32, 16)

    dsplats = [jnp.full((16,), d, jnp.int32) for d in range(TOK_DIM)]

    def transpose_chunk(buf):
        for j in range(CL):
            base = (buf * CL + j) * BW

            def tbody(tb, c):
                tidx = iota16 + (base + tb * 16)
                t0 = tb * 16
                vs = [plsc.load_gather(rows_v, [tidx, dsplats[d]])
                      for d in range(TOK_DIM)]
                for d in range(TOK_DIM):
                    s_v[buf, j, d // 8, d % 8, pl.ds(t0, 16)] = vs[d]
                return c

            lax.fori_loop(0, BW // 16, tbody, 0)

    idx_load(0, True)
    pl.when(lax.axis_index("s") == 0)(
        lambda: pltpu.sync_copy(posp_hbm, shpos))
    idx_load(0, False)
    gather(0, True)
    idx_load(1, True)
    plsc.subcore_barrier()

    def main(g, carry):
        gather(g, False)
        pl.when(g + 1 < NCH)(lambda: idx_load(g + 1, False))
        pl.when(g + 1 < NCH)(lambda: gather(g + 1, True))
        pl.when(g + 2 < NCH)(lambda: idx_load(g + 2, True))
        pl.when(g >= 1)(lambda: pos_out(g - 1, False))
        pl.when(g >= 2)(lambda: store(g - 2, False))
        transpose_chunk(lax.rem(g, NBUF))
        store(g, True)
        pos_out(g, True)
        return carry

    lax.fori_loop(0, NCH, main, 0)

    store(NCH - 2, False)
    store(NCH - 1, False)
    pos_out(NCH - 1, False)


def kernel(indices, tok_table, pos_table):
    tok_p = jnp.pad(tok_table, ((0, 0), (0, 8)))
    idxt = indices.T.astype(jnp.int32)
    posp = jnp.broadcast_to(
        pos_table.reshape(MAX_LEN, POS_DIM, 1),
        (MAX_LEN, POS_DIM, BW))
    out5 = _emb(tok_p, idxt, posp)
    return out5.transpose(2, 4, 0, 1, 3).reshape(BATCH, MAX_LEN, OUT_DIM)
